# Initial kernel scaffold; baseline (speedup 1.0000x reference)
import jax
import jax.numpy as jnp
from jax.experimental import pallas as pl


def kernel(x, edge_index, params):
    raise NotImplementedError("write your pallas kernel here")


# speedup vs baseline: 3.7697x; 3.7697x over previous
import functools

import jax
import jax.numpy as jnp
from jax import lax
from jax.experimental import pallas as pl
from jax.experimental.pallas import tpu as pltpu
from jax.experimental.pallas import tpu_sc as plsc

N_NODES = 10000
N_EDGES = 160000
E_TOT = N_EDGES + N_NODES
HEADS = 4
NEG_SLOPE = 0.2
NPAD = 10240
NW = 32
DPW = NPAD // NW
START_LEN = NW * DPW + 16
SRC_LEN = ((E_TOT + 31) // 32) * 32 + 32


def _mm_pair(h, wl, bl, wr, br):
    m, k = h.shape
    n = wl.shape[1]
    rb = 1024
    cb = min(2048, n)

    def body(x_ref, wl_ref, bl_ref, wr_ref, br_ref, ol_ref, or_ref):
        x = x_ref[...]
        ol_ref[...] = (
            jnp.dot(x, wl_ref[...], preferred_element_type=jnp.float32)
            + bl_ref[...]
        )
        or_ref[...] = (
            jnp.dot(x, wr_ref[...], preferred_element_type=jnp.float32)
            + br_ref[...]
        )

    out = pl.pallas_call(
        body,
        grid=(m // rb, n // cb),
        in_specs=[
            pl.BlockSpec((rb, k), lambda i, j: (i, 0)),
            pl.BlockSpec((k, cb), lambda i, j: (0, j)),
            pl.BlockSpec((1, cb), lambda i, j: (0, j)),
            pl.BlockSpec((k, cb), lambda i, j: (0, j)),
            pl.BlockSpec((1, cb), lambda i, j: (0, j)),
        ],
        out_specs=[
            pl.BlockSpec((rb, cb), lambda i, j: (i, j)),
            pl.BlockSpec((rb, cb), lambda i, j: (i, j)),
        ],
        out_shape=[jax.ShapeDtypeStruct((m, n), jnp.float32)] * 2,
    )(h, wl, bl, wr, br)
    return out


def _log_softmax2(h):
    def body(x_ref, o_ref):
        x = x_ref[...]
        x0 = x[:, 0:1]
        x1 = x[:, 1:2]
        mx = jnp.maximum(x0, x1)
        ls = mx + jnp.log(jnp.exp(x0 - mx) + jnp.exp(x1 - mx))
        o_ref[...] = x - ls

    return pl.pallas_call(
        body,
        grid=(NPAD // 1024,),
        in_specs=[pl.BlockSpec((1024, 32), lambda i: (i, 0))],
        out_specs=pl.BlockSpec((1024, 32), lambda i: (i, 0)),
        out_shape=jax.ShapeDtypeStruct((NPAD, 32), jnp.float32),
    )(h)


@functools.lru_cache(maxsize=None)
def _make_edge_kernel(cp: int, do_relu: bool):
    hcp = HEADS * cp
    nbc = cp // 16
    info = plsc.get_sparse_core_info()
    nc = info.num_cores
    mesh = plsc.VectorSubcoreMesh(core_axis_name="c", subcore_axis_name="s")
    a_pos = 0.5 * (1.0 + NEG_SLOPE)
    a_neg = 0.5 * (1.0 - NEG_SLOPE)

    @functools.partial(
        pl.kernel,
        out_type=jax.ShapeDtypeStruct((NPAD, cp), jnp.float32),
        mesh=mesh,
        compiler_params=pltpu.CompilerParams(needs_layout_passes=False),
        scratch_types=[
            pltpu.VMEM((DPW + 16,), jnp.int32),
            pltpu.VMEM((HEADS, cp), jnp.float32),
            pltpu.VMEM((cp,), jnp.float32),
            pltpu.VMEM((32,), jnp.int32),
            pltpu.VMEM((16, hcp), jnp.float32),
            pltpu.VMEM((hcp,), jnp.float32),
            pltpu.VMEM((hcp,), jnp.float32),
            pltpu.VMEM((cp,), jnp.float32),
            pltpu.VMEM((256,), jnp.float32),
            pltpu.SemaphoreType.DMA,
        ],
    )
    def ek(xl_hbm, xr_hbm, src_hbm, start_hbm, att_hbm, b_hbm, out_hbm,
           startv, attv, biasv, idxbuf, xlg, xrv, accv, outrow, partsv,
           sem):
        wid = lax.axis_index("s") * nc + lax.axis_index("c")
        d0 = wid * DPW
        pltpu.sync_copy(start_hbm.at[pl.ds(d0, DPW + 16)], startv)
        pltpu.sync_copy(att_hbm, attv)
        pltpu.sync_copy(b_hbm, biasv)
        lane = lax.iota(jnp.int32, 16)
        dnums = lax.GatherDimensionNumbers(
            offset_dims=(), collapsed_slice_dims=(0,), start_index_map=(0,))

        def _shuf(v, idx):
            return lax.gather(
                v, idx.reshape(16, 1), dnums, slice_sizes=(1,),
                mode=lax.GatherScatterMode.PROMISE_IN_BOUNDS)

        def _bsum(v):
            for kk in (8, 4, 2, 1):
                v = v + _shuf(v, lane ^ kk)
            return v

        def _bmax(v):
            for kk in (8, 4, 2, 1):
                v = jnp.maximum(v, _shuf(v, lane ^ kk))
            return v

        def dst_body(dloc, _):
            sv = startv[pl.ds(dloc, 16)]
            e0 = sv[0]
            e1 = sv[1]
            g = e1 - e0
            d = d0 + dloc
            pltpu.sync_copy(xr_hbm.at[d], xrv)

            def zacc(i, c):
                accv[pl.ds(i * 16, 16)] = jnp.zeros((16,), jnp.float32)
                return c

            lax.fori_loop(0, hcp // 16, zacc, 0)
            nb = (g + 15) // 16
            neg = jnp.full((16,), -1e30, jnp.float32)
            zero = jnp.zeros((16,), jnp.float32)

            def blk(b, carry):
                ms, ss = carry
                eb = e0 + b * 16
                eba = pl.multiple_of((eb // 8) * 8, 8)
                off = eb - eba
                pltpu.sync_copy(src_hbm.at[pl.ds(eba, 32)], idxbuf)
                rem = g - b * 16
                idx = idxbuf[pl.ds(off, 16)]
                live = lane < rem
                idx = jnp.where(live, idx, 0)
                pltpu.async_copy(xl_hbm.at[idx], xlg, sem).wait()
                new_ms = []
                new_ss = []
                for h in range(HEADS):
                    hb = h * cp

                    def cbody(cb, parts):
                        base = hb + cb * 16
                        xr_b = xrv[pl.ds(base, 16)]
                        at_b = attv[h, pl.ds(cb * 16, 16)]
                        ap = at_b * a_pos
                        an = at_b * a_neg
                        out = []
                        for j in range(16):
                            t = xlg[j, pl.ds(base, 16)] + xr_b
                            out.append(parts[j] + ap * t + an * jnp.abs(t))
                        return tuple(out)

                    parts = lax.fori_loop(
                        0, nbc, cbody,
                        tuple(jnp.zeros((16,), jnp.float32)
                              for _ in range(16)))
                    for j in range(16):
                        partsv[pl.ds(j * 16, 16)] = parts[j]
                    logits = jnp.zeros((16,), jnp.float32)
                    for cc in range(16):
                        logits = logits + plsc.load_gather(
                            partsv, [lane * 16 + cc])
                    logits = jnp.where(live, logits, neg)
                    m_new = jnp.maximum(ms[h], _bmax(logits))
                    scale = jnp.exp(ms[h] - m_new)
                    w = jnp.exp(logits - m_new)
                    w = jnp.where(live, w, zero)
                    s_new = ss[h] * scale + _bsum(w)
                    wj = [
                        _shuf(w, jnp.full((16,), j, jnp.int32))
                        for j in range(16)
                    ]

                    def abody(cb, c):
                        base = hb + cb * 16
                        a = accv[pl.ds(base, 16)] * scale
                        for j in range(16):
                            a = a + wj[j] * xlg[j, pl.ds(base, 16)]
                        accv[pl.ds(base, 16)] = a
                        return c

                    lax.fori_loop(0, nbc, abody, 0)
                    new_ms.append(m_new)
                    new_ss.append(s_new)
                return tuple(new_ms), tuple(new_ss)

            init_m = tuple(jnp.full((16,), -1e30, jnp.float32)
                           for _ in range(HEADS))
            init_s = tuple(jnp.zeros((16,), jnp.float32)
                           for _ in range(HEADS))
            ms, ss = lax.fori_loop(0, nb, blk, (init_m, init_s))
            inv = [1.0 / (ss[h] + 1e-16) for h in range(HEADS)]

            def obody(cb, c):
                base = cb * 16
                r = biasv[pl.ds(base, 16)]
                for h in range(HEADS):
                    r = r + 0.25 * inv[h] * accv[pl.ds(h * cp + base, 16)]
                if do_relu:
                    r = jnp.maximum(r, 0.0)
                outrow[pl.ds(base, 16)] = r
                return c

            lax.fori_loop(0, nbc, obody, 0)
            pltpu.sync_copy(outrow, out_hbm.at[d])
            return 0

        lax.fori_loop(0, DPW, dst_body, 0)

    return ek


def _prep_params(params):
    out = []
    for li, p in enumerate(params):
        fi, hfo = p["Wl"].shape
        fo = hfo // HEADS
        cp = max(fo, 32)
        fi_pad = 8 if li == 0 else fi

        def padw(w, fi=fi, fo=fo, cp=cp, fi_pad=fi_pad):
            w = w.reshape(fi, HEADS, fo)
            w = jnp.pad(w, ((0, fi_pad - fi), (0, 0), (0, cp - fo)))
            return w.reshape(fi_pad, HEADS * cp)

        def padb(b, fo=fo, cp=cp):
            b = b.reshape(HEADS, fo)
            b = jnp.pad(b, ((0, 0), (0, cp - fo)))
            return b.reshape(1, HEADS * cp)

        out.append({
            "cp": cp,
            "Wl": padw(p["Wl"]),
            "bl": padb(p["bl"]),
            "Wr": padw(p["Wr"]),
            "br": padb(p["br"]),
            "att": jnp.pad(p["att"], ((0, 0), (0, cp - fo))),
            "b": jnp.pad(p["b"], (0, cp - fo)),
        })
    return out


def kernel(x, edge_index, params):
    loops = jnp.arange(N_NODES, dtype=edge_index.dtype)
    src = jnp.concatenate([edge_index[0], loops])
    dst = jnp.concatenate([edge_index[1], loops])
    perm = jnp.argsort(dst)
    src_s = src[perm].astype(jnp.int32)
    dst_s = dst[perm].astype(jnp.int32)
    start = jnp.searchsorted(
        dst_s, jnp.arange(START_LEN, dtype=jnp.int32)).astype(jnp.int32)
    src_pad = jnp.pad(src_s, (0, SRC_LEN - E_TOT))

    h = jnp.pad(x.astype(jnp.float32),
                ((0, NPAD - N_NODES), (0, 8 - x.shape[1])))
    pp = _prep_params(params)
    for li, p in enumerate(pp):
        xl, xr = _mm_pair(h, p["Wl"], p["bl"], p["Wr"], p["br"])
        ek = _make_edge_kernel(p["cp"], li < len(pp) - 1)
        h = ek(xl, xr, src_pad, start, p["att"], p["b"])
    out = _log_softmax2(h)
    return out[:N_NODES, :2]
